# baseline (speedup 1.0000x reference)
import functools

import jax
import jax.numpy as jnp
from jax import lax
from jax.experimental import pallas as pl
from jax.experimental.pallas import tpu as pltpu
from jax.experimental.pallas import tpu_sc as plsc

N_NODES = 10000
N_EDGES = 320000
D = 128

NC = 2
NS = 16
NW = NC * NS
CH = 128
NFULL = 79
EPW = NFULL * CH
E_PAD = NW * EPW
N_PAD = 10016
ROWS_PT = 624
ROWS_LAST = N_PAD - (NS - 1) * ROWS_PT
CNT_PT = 624
NB = 2
NIB = 4


@functools.cache
def _make_seg_sum(with_cnt: bool):
  mesh = plsc.VectorSubcoreMesh(
      core_axis_name="c", subcore_axis_name="s", num_cores=NC,
      num_subcores=NS)

  out_type = [jax.ShapeDtypeStruct((NC, N_PAD, D), jnp.float32)]
  if with_cnt:
    out_type.append(jax.ShapeDtypeStruct((NC * N_PAD,), jnp.float32))

  scratch = dict(
      sbuf=[pltpu.VMEM((CH,), jnp.int32) for _ in range(NIB)],
      dbuf=[pltpu.VMEM((CH,), jnp.int32) for _ in range(NIB)],
      rows=[pltpu.VMEM((CH, D), jnp.float32) for _ in range(NB)],
      ones_v=pltpu.VMEM((CH,), jnp.float32),
      cbuf=pltpu.VMEM((ROWS_LAST,), jnp.float32),
      acc_sp=pltpu.VMEM_SHARED((N_PAD, D), jnp.float32),
      cnt_sp=pltpu.VMEM_SHARED((N_PAD,), jnp.float32),
      sem_i=[pltpu.SemaphoreType.DMA for _ in range(NIB)],
      sem_g=[pltpu.SemaphoreType.DMA for _ in range(NB)],
      sem_s=[pltpu.SemaphoreType.DMA for _ in range(NB)],
      sem_c=[pltpu.SemaphoreType.DMA for _ in range(NB)],
  )

  def body(src_hbm, dst_hbm, feat_hbm, z2d_hbm, *outs, sbuf, dbuf, rows,
           ones_v, cbuf, acc_sp, cnt_sp, sem_i, sem_g, sem_s, sem_c):
    if with_cnt:
      acc_out, cnt_out = outs
    else:
      (acc_out,) = outs

    cid = lax.axis_index("c")
    sid = lax.axis_index("s")
    wid = cid * NS + sid

    @pl.when(sid < NS - 1)
    def _():
      pltpu.sync_copy(z2d_hbm.at[pl.ds(sid * ROWS_PT, ROWS_PT)],
                      acc_sp.at[pl.ds(sid * ROWS_PT, ROWS_PT)])

    @pl.when(sid == NS - 1)
    def _():
      pltpu.sync_copy(z2d_hbm.at[pl.ds((NS - 1) * ROWS_PT, ROWS_LAST)],
                      acc_sp.at[pl.ds((NS - 1) * ROWS_PT, ROWS_LAST)])

    if with_cnt:
      for j in range(CH // 16):
        ones_v[pl.ds(j * 16, 16)] = jnp.ones((16,), jnp.float32)
      for j in range(ROWS_LAST // 16):
        cbuf[pl.ds(j * 16, 16)] = jnp.zeros((16,), jnp.float32)

      @pl.when(sid < NS - 1)
      def _():
        pltpu.sync_copy(cbuf.at[pl.ds(0, CNT_PT)],
                        cnt_sp.at[pl.ds(sid * CNT_PT, CNT_PT)])

      @pl.when(sid == NS - 1)
      def _():
        nlast = N_PAD - (NS - 1) * CNT_PT
        pltpu.sync_copy(cbuf.at[pl.ds(0, nlast)],
                        cnt_sp.at[pl.ds((NS - 1) * CNT_PT, nlast)])

    plsc.subcore_barrier()

    ebase = wid * EPW

    def cbase(i):
      return jnp.minimum(ebase + i * CH, E_PAD - CH)

    def idx_start(i, ib):
      base = cbase(i)
      pltpu.async_copy(src_hbm.at[pl.ds(base, CH)], sbuf[ib], sem_i[ib])
      pltpu.async_copy(dst_hbm.at[pl.ds(base, CH)], dbuf[ib], sem_i[ib])

    def idx_wait(ib):
      pltpu.make_async_copy(src_hbm.at[pl.ds(0, CH)], sbuf[ib],
                            sem_i[ib]).wait()
      pltpu.make_async_copy(dst_hbm.at[pl.ds(0, CH)], dbuf[ib],
                            sem_i[ib]).wait()

    def gather_start(b, ib):
      pltpu.async_copy(feat_hbm.at[sbuf[ib]], rows[b], sem_g[b])

    def gather_wait(b, ib):
      pltpu.make_async_copy(feat_hbm.at[sbuf[ib]], rows[b],
                            sem_g[b]).wait()

    def scatter_start(b, ib):
      pltpu.async_copy(rows[b], acc_sp.at[dbuf[ib]], sem_s[b], add=True)
      if with_cnt:
        pltpu.async_copy(ones_v, cnt_sp.at[dbuf[ib]], sem_c[b], add=True)

    def scatter_wait(b, ib):
      pltpu.make_async_copy(rows[b], acc_sp.at[dbuf[ib]], sem_s[b]).wait()
      if with_cnt:
        pltpu.make_async_copy(ones_v, cnt_sp.at[dbuf[ib]],
                              sem_c[b]).wait()

    def step(i, b, ib, first=False):
      nb = (b + 1) % NB
      nib = (ib + 1) % NIB
      if not first:
        scatter_wait(nb, (ib + NIB - 1) % NIB)
      idx_wait(nib)
      gather_start(nb, nib)
      gather_wait(b, ib)
      scatter_start(b, ib)
      idx_start(i + 2, (ib + 2) % NIB)

    idx_start(0, 0)
    idx_wait(0)
    gather_start(0, 0)
    idx_start(1, 1)

    step(0, 0, 0, first=True)

    def group(j, _):
      i0 = 4 * j + 1
      for k in range(4):
        i = i0 + k
        step(i, (1 + k) % NB, (1 + k) % NIB)
      return 0

    n_grp = (NFULL - 1) // 4
    lax.fori_loop(0, n_grp, group, 0)

    for i in range(1 + 4 * n_grp, NFULL):
      step(i, i % NB, i % NIB)

    scatter_wait((NFULL - 1) % NB, (NFULL - 1) % NIB)
    gather_wait(NFULL % NB, NFULL % NIB)
    idx_wait((NFULL + 1) % NIB)

    plsc.subcore_barrier()

    @pl.when(sid < NS - 1)
    def _():
      pltpu.sync_copy(acc_sp.at[pl.ds(sid * ROWS_PT, ROWS_PT)],
                      acc_out.at[cid, pl.ds(sid * ROWS_PT, ROWS_PT)])

    @pl.when(sid == NS - 1)
    def _():
      pltpu.sync_copy(acc_sp.at[pl.ds((NS - 1) * ROWS_PT, ROWS_LAST)],
                      acc_out.at[cid, pl.ds((NS - 1) * ROWS_PT, ROWS_LAST)])

    if with_cnt:
      @pl.when(sid < NS - 1)
      def _():
        pltpu.sync_copy(cnt_sp.at[pl.ds(sid * CNT_PT, CNT_PT)],
                        cbuf.at[pl.ds(0, CNT_PT)])
        pltpu.sync_copy(
            cbuf.at[pl.ds(0, CNT_PT)],
            cnt_out.at[pl.ds(cid * N_PAD + sid * CNT_PT, CNT_PT)])

      @pl.when(sid == NS - 1)
      def _():
        nlast = N_PAD - (NS - 1) * CNT_PT
        pltpu.sync_copy(cnt_sp.at[pl.ds((NS - 1) * CNT_PT, nlast)],
                        cbuf.at[pl.ds(0, nlast)])
        pltpu.sync_copy(
            cbuf.at[pl.ds(0, nlast)],
            cnt_out.at[pl.ds(cid * N_PAD + (NS - 1) * CNT_PT, nlast)])

  return pl.kernel(body, out_type=out_type, mesh=mesh,
                   scratch_types=scratch,
                   name="seg_sum_cnt" if with_cnt else "seg_sum")


RB = 2504
GRID = N_PAD // RB


def _mm_bias_body(x_ref, w_ref, b_ref, o_ref):
  o_ref[...] = (jnp.dot(x_ref[...], w_ref[...],
                        preferred_element_type=jnp.float32) + b_ref[...])


def _mm_bias(x, w, b):
  return pl.pallas_call(
      _mm_bias_body,
      grid=(GRID,),
      in_specs=[
          pl.BlockSpec((RB, D), lambda i: (i, 0)),
          pl.BlockSpec((D, D), lambda i: (0, 0)),
          pl.BlockSpec((D,), lambda i: (0,)),
      ],
      out_specs=pl.BlockSpec((RB, D), lambda i: (i, 0)),
      out_shape=jax.ShapeDtypeStruct((N_PAD, D), jnp.float32),
  )(x, w, b)


def _combine_body(acc_ref, cnt_ref, xr_ref, wl_ref, o_ref, *, final: bool):
  s = acc_ref[0] + acc_ref[1]
  c = cnt_ref[0] + cnt_ref[1]
  mean = s / jnp.maximum(c, 1.0)
  z = (jnp.dot(mean, wl_ref[...], preferred_element_type=jnp.float32)
       + xr_ref[...])
  if final:
    m = jnp.max(z, axis=1, keepdims=True)
    e = jnp.exp(z - m)
    lse = jnp.log(jnp.sum(e, axis=1, keepdims=True)) + m
    o_ref[...] = z - lse
  else:
    o_ref[...] = jnp.maximum(z, 0.0)


def _combine(acc, cnt, xr, w_l, final: bool):
  return pl.pallas_call(
      functools.partial(_combine_body, final=final),
      grid=(GRID,),
      in_specs=[
          pl.BlockSpec((NC, RB, D), lambda i: (0, i, 0)),
          pl.BlockSpec((NC, RB, 1), lambda i: (0, i, 0)),
          pl.BlockSpec((RB, D), lambda i: (i, 0)),
          pl.BlockSpec((D, D), lambda i: (0, 0)),
      ],
      out_specs=pl.BlockSpec((RB, D), lambda i: (i, 0)),
      out_shape=jax.ShapeDtypeStruct((N_PAD, D), jnp.float32),
  )(acc, cnt, xr, w_l)


def kernel(x, edge_index, W1_l, W1_r, b1, W2_l, W2_r, b2):
  ei = edge_index.astype(jnp.int32)
  pad = jnp.full((E_PAD - N_EDGES,), N_NODES, jnp.int32)
  src = jnp.concatenate([ei[0], pad])
  dst = jnp.concatenate([ei[1], pad])
  xp = jnp.concatenate([x, jnp.zeros((N_PAD - N_NODES, D), jnp.float32)])
  z2d = jnp.zeros((N_PAD, D), jnp.float32)

  xr = _mm_bias(xp, W1_r, b1)
  acc1, cnt = _make_seg_sum(True)(src, dst, xp, z2d)
  cnt3 = cnt.reshape(NC, N_PAD, 1)
  h = _combine(acc1, cnt3, xr, W1_l, final=False)
  hr = _mm_bias(h, W2_r, b2)
  (acc2,) = _make_seg_sum(False)(src, dst, h, z2d)
  out = _combine(acc2, cnt3, hr, W2_l, final=True)
  return out[:N_NODES]


# speedup vs baseline: 1.9047x; 1.9047x over previous
import functools

import jax
import jax.numpy as jnp
from jax import lax
from jax.experimental import pallas as pl
from jax.experimental.pallas import tpu as pltpu
from jax.experimental.pallas import tpu_sc as plsc

N_NODES = 10000
N_EDGES = 320000
D = 128

NC = 2
NS = 16
NW = NC * NS
CH = 128
NFULL = 79
EPW = NFULL * CH
E_PAD = NW * EPW
N_PAD = 10112
ROWS_PT = 632
ROWS_LAST = N_PAD - (NS - 1) * ROWS_PT
CNT_PT = 632
NB = 2
NIB = 4


@functools.cache
def _make_seg_sum(with_cnt: bool):
  mesh = plsc.VectorSubcoreMesh(
      core_axis_name="c", subcore_axis_name="s", num_cores=NC,
      num_subcores=NS)

  out_type = [jax.ShapeDtypeStruct((NC, N_PAD, D), jnp.float32)]
  if with_cnt:
    out_type.append(jax.ShapeDtypeStruct((NC * N_PAD,), jnp.float32))

  scratch = dict(
      sbuf=[pltpu.VMEM((CH,), jnp.int32) for _ in range(NIB)],
      dbuf=[pltpu.VMEM((CH,), jnp.int32) for _ in range(NIB)],
      rows=[pltpu.VMEM((CH, D), jnp.float32) for _ in range(NB)],
      ones_v=pltpu.VMEM((CH,), jnp.float32),
      cbuf=pltpu.VMEM((640,), jnp.float32),
      acc_sp=pltpu.VMEM_SHARED((N_PAD, D), jnp.float32),
      cnt_sp=pltpu.VMEM_SHARED((N_PAD,), jnp.float32),
      sem_i=[pltpu.SemaphoreType.DMA for _ in range(NIB)],
      sem_g=[pltpu.SemaphoreType.DMA for _ in range(NB)],
      sem_s=[pltpu.SemaphoreType.DMA for _ in range(NB)],
      sem_c=[pltpu.SemaphoreType.DMA for _ in range(NB)],
  )

  def body(src_hbm, dst_hbm, feat_hbm, z2d_hbm, *outs, sbuf, dbuf, rows,
           ones_v, cbuf, acc_sp, cnt_sp, sem_i, sem_g, sem_s, sem_c):
    if with_cnt:
      acc_out, cnt_out = outs
    else:
      (acc_out,) = outs

    cid = lax.axis_index("c")
    sid = lax.axis_index("s")
    wid = cid * NS + sid

    @pl.when(sid < NS - 1)
    def _():
      pltpu.sync_copy(z2d_hbm.at[pl.ds(sid * ROWS_PT, ROWS_PT)],
                      acc_sp.at[pl.ds(sid * ROWS_PT, ROWS_PT)])

    @pl.when(sid == NS - 1)
    def _():
      pltpu.sync_copy(z2d_hbm.at[pl.ds((NS - 1) * ROWS_PT, ROWS_LAST)],
                      acc_sp.at[pl.ds((NS - 1) * ROWS_PT, ROWS_LAST)])

    if with_cnt:
      for j in range(CH // 16):
        ones_v[pl.ds(j * 16, 16)] = jnp.ones((16,), jnp.float32)
      for j in range(640 // 16):
        cbuf[pl.ds(j * 16, 16)] = jnp.zeros((16,), jnp.float32)

      @pl.when(sid < NS - 1)
      def _():
        pltpu.sync_copy(cbuf.at[pl.ds(0, CNT_PT)],
                        cnt_sp.at[pl.ds(sid * CNT_PT, CNT_PT)])

      @pl.when(sid == NS - 1)
      def _():
        nlast = N_PAD - (NS - 1) * CNT_PT
        pltpu.sync_copy(cbuf.at[pl.ds(0, nlast)],
                        cnt_sp.at[pl.ds((NS - 1) * CNT_PT, nlast)])

    plsc.subcore_barrier()

    ebase = wid * EPW

    def cbase(i):
      return jnp.minimum(ebase + i * CH, E_PAD - CH)

    def idx_start(i, ib):
      base = cbase(i)
      pltpu.async_copy(src_hbm.at[pl.ds(base, CH)], sbuf[ib], sem_i[ib])
      pltpu.async_copy(dst_hbm.at[pl.ds(base, CH)], dbuf[ib], sem_i[ib])

    def idx_wait(ib):
      pltpu.make_async_copy(src_hbm.at[pl.ds(0, CH)], sbuf[ib],
                            sem_i[ib]).wait()
      pltpu.make_async_copy(dst_hbm.at[pl.ds(0, CH)], dbuf[ib],
                            sem_i[ib]).wait()

    def gather_start(b, ib):
      pltpu.async_copy(feat_hbm.at[sbuf[ib]], rows[b], sem_g[b])

    def gather_wait(b, ib):
      pltpu.make_async_copy(feat_hbm.at[sbuf[ib]], rows[b],
                            sem_g[b]).wait()

    def scatter_start(b, ib):
      pltpu.async_copy(rows[b], acc_sp.at[dbuf[ib]], sem_s[b], add=True)
      if with_cnt:
        pltpu.async_copy(ones_v, cnt_sp.at[dbuf[ib]], sem_c[b], add=True)

    def scatter_wait(b, ib):
      pltpu.make_async_copy(rows[b], acc_sp.at[dbuf[ib]], sem_s[b]).wait()
      if with_cnt:
        pltpu.make_async_copy(ones_v, cnt_sp.at[dbuf[ib]],
                              sem_c[b]).wait()

    def step(i, b, ib, first=False):
      nb = (b + 1) % NB
      nib = (ib + 1) % NIB
      if not first:
        scatter_wait(nb, (ib + NIB - 1) % NIB)
      idx_wait(nib)
      gather_start(nb, nib)
      gather_wait(b, ib)
      scatter_start(b, ib)
      idx_start(i + 2, (ib + 2) % NIB)

    idx_start(0, 0)
    idx_wait(0)
    gather_start(0, 0)
    idx_start(1, 1)

    step(0, 0, 0, first=True)

    def group(j, _):
      i0 = 4 * j + 1
      for k in range(4):
        i = i0 + k
        step(i, (1 + k) % NB, (1 + k) % NIB)
      return 0

    n_grp = (NFULL - 1) // 4
    lax.fori_loop(0, n_grp, group, 0)

    for i in range(1 + 4 * n_grp, NFULL):
      step(i, i % NB, i % NIB)

    scatter_wait((NFULL - 1) % NB, (NFULL - 1) % NIB)
    gather_wait(NFULL % NB, NFULL % NIB)
    idx_wait((NFULL + 1) % NIB)

    plsc.subcore_barrier()

    @pl.when(sid < NS - 1)
    def _():
      pltpu.sync_copy(acc_sp.at[pl.ds(sid * ROWS_PT, ROWS_PT)],
                      acc_out.at[cid, pl.ds(sid * ROWS_PT, ROWS_PT)])

    @pl.when(sid == NS - 1)
    def _():
      pltpu.sync_copy(acc_sp.at[pl.ds((NS - 1) * ROWS_PT, ROWS_LAST)],
                      acc_out.at[cid, pl.ds((NS - 1) * ROWS_PT, ROWS_LAST)])

    if with_cnt:
      @pl.when(sid < NS - 1)
      def _():
        pltpu.sync_copy(cnt_sp.at[pl.ds(sid * CNT_PT, CNT_PT)],
                        cbuf.at[pl.ds(0, CNT_PT)])
        pltpu.sync_copy(
            cbuf.at[pl.ds(0, CNT_PT)],
            cnt_out.at[pl.ds(cid * N_PAD + sid * CNT_PT, CNT_PT)])

      @pl.when(sid == NS - 1)
      def _():
        nlast = N_PAD - (NS - 1) * CNT_PT
        pltpu.sync_copy(cnt_sp.at[pl.ds((NS - 1) * CNT_PT, nlast)],
                        cbuf.at[pl.ds(0, nlast)])
        pltpu.sync_copy(
            cbuf.at[pl.ds(0, nlast)],
            cnt_out.at[pl.ds(cid * N_PAD + (NS - 1) * CNT_PT, nlast)])

  return pl.kernel(body, out_type=out_type, mesh=mesh,
                   scratch_types=scratch,
                   name="seg_sum_cnt" if with_cnt else "seg_sum")


RB = 1264
GRID = N_PAD // RB


def _mm_bias_body(x_ref, w_ref, b_ref, o_ref):
  o_ref[...] = (jnp.dot(x_ref[...], w_ref[...],
                        preferred_element_type=jnp.float32) + b_ref[...])


def _mm_bias(x, w, b):
  return pl.pallas_call(
      _mm_bias_body,
      grid=(GRID,),
      in_specs=[
          pl.BlockSpec((RB, D), lambda i: (i, 0)),
          pl.BlockSpec((D, D), lambda i: (0, 0)),
          pl.BlockSpec((D,), lambda i: (0,)),
      ],
      out_specs=pl.BlockSpec((RB, D), lambda i: (i, 0)),
      out_shape=jax.ShapeDtypeStruct((N_PAD, D), jnp.float32),
  )(x, w, b)


def _combine_body(acc_ref, cnt_ref, xr_ref, wl_ref, o_ref, *, final: bool):
  s = acc_ref[0] + acc_ref[1]
  c = cnt_ref[0] + cnt_ref[1]
  mean = s / jnp.maximum(c, 1.0)
  z = (jnp.dot(mean, wl_ref[...], preferred_element_type=jnp.float32)
       + xr_ref[...])
  if final:
    m = jnp.max(z, axis=1, keepdims=True)
    e = jnp.exp(z - m)
    lse = jnp.log(jnp.sum(e, axis=1, keepdims=True)) + m
    o_ref[...] = z - lse
  else:
    o_ref[...] = jnp.maximum(z, 0.0)


def _combine(acc, cnt, xr, w_l, final: bool):
  return pl.pallas_call(
      functools.partial(_combine_body, final=final),
      grid=(GRID,),
      in_specs=[
          pl.BlockSpec((NC, RB, D), lambda i: (0, i, 0)),
          pl.BlockSpec((NC, RB, 1), lambda i: (0, i, 0)),
          pl.BlockSpec((RB, D), lambda i: (i, 0)),
          pl.BlockSpec((D, D), lambda i: (0, 0)),
      ],
      out_specs=pl.BlockSpec((RB, D), lambda i: (i, 0)),
      out_shape=jax.ShapeDtypeStruct((N_PAD, D), jnp.float32),
  )(acc, cnt, xr, w_l)


def kernel(x, edge_index, W1_l, W1_r, b1, W2_l, W2_r, b2):
  ei = edge_index.astype(jnp.int32)
  ppw = EPW - N_EDGES // NW
  pad = jnp.broadcast_to(N_NODES + jnp.arange(ppw, dtype=jnp.int32),
                         (NW, ppw))
  src = jnp.concatenate([ei[0].reshape(NW, -1), pad], axis=1).reshape(-1)
  dst = jnp.concatenate([ei[1].reshape(NW, -1), pad], axis=1).reshape(-1)
  xp = jnp.concatenate([x, jnp.zeros((N_PAD - N_NODES, D), jnp.float32)])
  z2d = jnp.zeros((N_PAD, D), jnp.float32)

  xr = _mm_bias(xp, W1_r, b1)
  acc1, cnt = _make_seg_sum(True)(src, dst, xp, z2d)
  cnt3 = cnt.reshape(NC, N_PAD, 1)
  h = _combine(acc1, cnt3, xr, W1_l, final=False)
  hr = _mm_bias(h, W2_r, b2)
  (acc2,) = _make_seg_sum(False)(src, dst, h, z2d)
  out = _combine(acc2, cnt3, hr, W2_l, final=True)
  return out[:N_NODES]


# speedup vs baseline: 1.9135x; 1.0046x over previous
import functools

import jax
import jax.numpy as jnp
from jax import lax
from jax.experimental import pallas as pl
from jax.experimental.pallas import tpu as pltpu
from jax.experimental.pallas import tpu_sc as plsc

N_NODES = 10000
N_EDGES = 320000
D = 128

NC = 2
NS = 16
NW = NC * NS
CH = 128
NFULL = 79
EPW = NFULL * CH
E_PAD = NW * EPW
N_PAD = 10112
ROWS_PT = 632
ROWS_LAST = N_PAD - (NS - 1) * ROWS_PT
CNT_PT = 632
NB = 2
NIB = 4


@functools.cache
def _make_seg_sum(with_cnt: bool):
  mesh = plsc.VectorSubcoreMesh(
      core_axis_name="c", subcore_axis_name="s", num_cores=NC,
      num_subcores=NS)

  out_type = [jax.ShapeDtypeStruct((NC, N_PAD, D), jnp.float32)]
  if with_cnt:
    out_type.append(jax.ShapeDtypeStruct((NC * N_PAD,), jnp.float32))

  scratch = dict(
      sbuf=[pltpu.VMEM((CH,), jnp.int32) for _ in range(NIB)],
      dbuf=[pltpu.VMEM((CH,), jnp.int32) for _ in range(NIB)],
      rows=[pltpu.VMEM((CH, D), jnp.float32) for _ in range(NB)],
      ones_v=pltpu.VMEM((CH,), jnp.float32),
      cbuf=pltpu.VMEM((640,), jnp.float32),
      acc_sp=pltpu.VMEM_SHARED((N_PAD, D), jnp.float32),
      cnt_sp=pltpu.VMEM_SHARED((N_PAD,), jnp.float32),
      sem_i=[pltpu.SemaphoreType.DMA for _ in range(NIB)],
      sem_g=[pltpu.SemaphoreType.DMA for _ in range(NB)],
      sem_s=[pltpu.SemaphoreType.DMA for _ in range(NB)],
      sem_c=[pltpu.SemaphoreType.DMA for _ in range(NB)],
  )

  def body(src_hbm, dst_hbm, feat_hbm, z2d_hbm, *outs, sbuf, dbuf, rows,
           ones_v, cbuf, acc_sp, cnt_sp, sem_i, sem_g, sem_s, sem_c):
    if with_cnt:
      acc_out, cnt_out = outs
    else:
      (acc_out,) = outs

    cid = lax.axis_index("c")
    sid = lax.axis_index("s")
    wid = cid * NS + sid

    @pl.when(sid < NS - 1)
    def _():
      pltpu.sync_copy(z2d_hbm.at[pl.ds(sid * ROWS_PT, ROWS_PT)],
                      acc_sp.at[pl.ds(sid * ROWS_PT, ROWS_PT)])

    @pl.when(sid == NS - 1)
    def _():
      pltpu.sync_copy(z2d_hbm.at[pl.ds((NS - 1) * ROWS_PT, ROWS_LAST)],
                      acc_sp.at[pl.ds((NS - 1) * ROWS_PT, ROWS_LAST)])

    if with_cnt:
      for j in range(CH // 16):
        ones_v[pl.ds(j * 16, 16)] = jnp.ones((16,), jnp.float32)
      for j in range(640 // 16):
        cbuf[pl.ds(j * 16, 16)] = jnp.zeros((16,), jnp.float32)

      @pl.when(sid < NS - 1)
      def _():
        pltpu.sync_copy(cbuf.at[pl.ds(0, CNT_PT)],
                        cnt_sp.at[pl.ds(sid * CNT_PT, CNT_PT)])

      @pl.when(sid == NS - 1)
      def _():
        nlast = N_PAD - (NS - 1) * CNT_PT
        pltpu.sync_copy(cbuf.at[pl.ds(0, nlast)],
                        cnt_sp.at[pl.ds((NS - 1) * CNT_PT, nlast)])

    plsc.subcore_barrier()

    ebase = wid * EPW

    def cbase(i):
      return jnp.minimum(ebase + i * CH, E_PAD - CH)

    def idx_start(i, ib):
      base = cbase(i)
      pltpu.async_copy(src_hbm.at[pl.ds(base, CH)], sbuf[ib], sem_i[ib])
      pltpu.async_copy(dst_hbm.at[pl.ds(base, CH)], dbuf[ib], sem_i[ib])

    def idx_wait(ib):
      pltpu.make_async_copy(src_hbm.at[pl.ds(0, CH)], sbuf[ib],
                            sem_i[ib]).wait()
      pltpu.make_async_copy(dst_hbm.at[pl.ds(0, CH)], dbuf[ib],
                            sem_i[ib]).wait()

    def gather_start(b, ib):
      pltpu.async_copy(feat_hbm.at[sbuf[ib]], rows[b], sem_g[b])

    def gather_wait(b, ib):
      pltpu.make_async_copy(feat_hbm.at[sbuf[ib]], rows[b],
                            sem_g[b]).wait()

    def scatter_start(b, ib):
      pltpu.async_copy(rows[b], acc_sp.at[dbuf[ib]], sem_s[b], add=True)
      if with_cnt:
        pltpu.async_copy(ones_v, cnt_sp.at[dbuf[ib]], sem_c[b], add=True)

    def scatter_wait(b, ib):
      pltpu.make_async_copy(rows[b], acc_sp.at[dbuf[ib]], sem_s[b]).wait()
      if with_cnt:
        pltpu.make_async_copy(ones_v, cnt_sp.at[dbuf[ib]],
                              sem_c[b]).wait()

    def step(i, b, ib, first=False):
      nb = (b + 1) % NB
      nib = (ib + 1) % NIB
      idx_start(i + 2, (ib + 2) % NIB)
      idx_wait(nib)
      if not first:
        scatter_wait(nb, (ib + NIB - 1) % NIB)
      gather_start(nb, nib)
      gather_wait(b, ib)
      scatter_start(b, ib)

    idx_start(0, 0)
    idx_wait(0)
    gather_start(0, 0)
    idx_start(1, 1)

    step(0, 0, 0, first=True)

    def group(j, _):
      i0 = 4 * j + 1
      for k in range(4):
        i = i0 + k
        step(i, (1 + k) % NB, (1 + k) % NIB)
      return 0

    n_grp = (NFULL - 1) // 4
    lax.fori_loop(0, n_grp, group, 0)

    for i in range(1 + 4 * n_grp, NFULL):
      step(i, i % NB, i % NIB)

    scatter_wait((NFULL - 1) % NB, (NFULL - 1) % NIB)
    gather_wait(NFULL % NB, NFULL % NIB)
    idx_wait((NFULL + 1) % NIB)

    plsc.subcore_barrier()

    @pl.when(sid < NS - 1)
    def _():
      pltpu.sync_copy(acc_sp.at[pl.ds(sid * ROWS_PT, ROWS_PT)],
                      acc_out.at[cid, pl.ds(sid * ROWS_PT, ROWS_PT)])

    @pl.when(sid == NS - 1)
    def _():
      pltpu.sync_copy(acc_sp.at[pl.ds((NS - 1) * ROWS_PT, ROWS_LAST)],
                      acc_out.at[cid, pl.ds((NS - 1) * ROWS_PT, ROWS_LAST)])

    if with_cnt:
      @pl.when(sid < NS - 1)
      def _():
        pltpu.sync_copy(cnt_sp.at[pl.ds(sid * CNT_PT, CNT_PT)],
                        cbuf.at[pl.ds(0, CNT_PT)])
        pltpu.sync_copy(
            cbuf.at[pl.ds(0, CNT_PT)],
            cnt_out.at[pl.ds(cid * N_PAD + sid * CNT_PT, CNT_PT)])

      @pl.when(sid == NS - 1)
      def _():
        nlast = N_PAD - (NS - 1) * CNT_PT
        pltpu.sync_copy(cnt_sp.at[pl.ds((NS - 1) * CNT_PT, nlast)],
                        cbuf.at[pl.ds(0, nlast)])
        pltpu.sync_copy(
            cbuf.at[pl.ds(0, nlast)],
            cnt_out.at[pl.ds(cid * N_PAD + (NS - 1) * CNT_PT, nlast)])

  return pl.kernel(body, out_type=out_type, mesh=mesh,
                   scratch_types=scratch,
                   name="seg_sum_cnt" if with_cnt else "seg_sum")


RB = 1264
GRID = N_PAD // RB


def _mm_bias_body(x_ref, w_ref, b_ref, o_ref):
  o_ref[...] = (jnp.dot(x_ref[...], w_ref[...],
                        preferred_element_type=jnp.float32) + b_ref[...])


def _mm_bias(x, w, b):
  return pl.pallas_call(
      _mm_bias_body,
      grid=(GRID,),
      in_specs=[
          pl.BlockSpec((RB, D), lambda i: (i, 0)),
          pl.BlockSpec((D, D), lambda i: (0, 0)),
          pl.BlockSpec((D,), lambda i: (0,)),
      ],
      out_specs=pl.BlockSpec((RB, D), lambda i: (i, 0)),
      out_shape=jax.ShapeDtypeStruct((N_PAD, D), jnp.float32),
  )(x, w, b)


def _combine_body(acc_ref, cnt_ref, xr_ref, wl_ref, o_ref, *, final: bool):
  s = acc_ref[0] + acc_ref[1]
  c = cnt_ref[0] + cnt_ref[1]
  mean = s / jnp.maximum(c, 1.0)
  z = (jnp.dot(mean, wl_ref[...], preferred_element_type=jnp.float32)
       + xr_ref[...])
  if final:
    m = jnp.max(z, axis=1, keepdims=True)
    e = jnp.exp(z - m)
    lse = jnp.log(jnp.sum(e, axis=1, keepdims=True)) + m
    o_ref[...] = z - lse
  else:
    o_ref[...] = jnp.maximum(z, 0.0)


def _combine(acc, cnt, xr, w_l, final: bool):
  return pl.pallas_call(
      functools.partial(_combine_body, final=final),
      grid=(GRID,),
      in_specs=[
          pl.BlockSpec((NC, RB, D), lambda i: (0, i, 0)),
          pl.BlockSpec((NC, RB, 1), lambda i: (0, i, 0)),
          pl.BlockSpec((RB, D), lambda i: (i, 0)),
          pl.BlockSpec((D, D), lambda i: (0, 0)),
      ],
      out_specs=pl.BlockSpec((RB, D), lambda i: (i, 0)),
      out_shape=jax.ShapeDtypeStruct((N_PAD, D), jnp.float32),
  )(acc, cnt, xr, w_l)


def kernel(x, edge_index, W1_l, W1_r, b1, W2_l, W2_r, b2):
  ei = edge_index.astype(jnp.int32)
  ppw = EPW - N_EDGES // NW
  pad = jnp.broadcast_to(N_NODES + jnp.arange(ppw, dtype=jnp.int32),
                         (NW, ppw))
  src = jnp.concatenate([ei[0].reshape(NW, -1), pad], axis=1).reshape(-1)
  dst = jnp.concatenate([ei[1].reshape(NW, -1), pad], axis=1).reshape(-1)
  xp = jnp.concatenate([x, jnp.zeros((N_PAD - N_NODES, D), jnp.float32)])
  z2d = jnp.zeros((N_PAD, D), jnp.float32)

  xr = _mm_bias(xp, W1_r, b1)
  acc1, cnt = _make_seg_sum(True)(src, dst, xp, z2d)
  cnt3 = cnt.reshape(NC, N_PAD, 1)
  h = _combine(acc1, cnt3, xr, W1_l, final=False)
  hr = _mm_bias(h, W2_r, b2)
  (acc2,) = _make_seg_sum(False)(src, dst, h, z2d)
  out = _combine(acc2, cnt3, hr, W2_l, final=True)
  return out[:N_NODES]
